# Initial kernel scaffold; baseline (speedup 1.0000x reference)
import jax
import jax.numpy as jnp
from jax.experimental import pallas as pl


def kernel(reals, card_nums, embed_table, cardIDs, actionIDs):
    raise NotImplementedError("write your pallas kernel here")


# speedup vs baseline: 4.1136x; 4.1136x over previous
import functools

import jax
import jax.numpy as jnp
from jax import lax
from jax.experimental import pallas as pl
from jax.experimental.pallas import tpu as pltpu
from jax.experimental.pallas import tpu_sc as plsc

_GRP = 128
_NW = 32
_B = 4096
_CARD_N = _B * 50
_ACT_N = _B * 20 * 8
_APW = _ACT_N // _NW
_AG = _APW // _GRP
_BATCH = 8
_ROWS = _BATCH * _GRP
_AI = _AG // _BATCH


def _sc_prepare(table, cids, aids, nums):
    mesh = plsc.VectorSubcoreMesh(core_axis_name="c", subcore_axis_name="s")

    @functools.partial(
        pl.kernel,
        mesh=mesh,
        out_type=[
            jax.ShapeDtypeStruct((_CARD_N, 80), jnp.float32),
            jax.ShapeDtypeStruct((_ACT_N, 64), jnp.float32),
        ],
        scratch_types=[
            pltpu.VMEM((_AG, _GRP), jnp.int32),
            pltpu.VMEM((_ROWS, 64), jnp.float32),
            pltpu.VMEM((_ROWS, 16), jnp.float32),
            pltpu.SemaphoreType.DMA,
        ],
        compiler_params=pltpu.CompilerParams(use_tc_tiling_on_sc=False),
    )
    def k(table_h, cids_h, aids_h, nums_h, outc_h, outa_h,
          idx_v, rows_v, nums_v, sem):
        w = lax.axis_index("s") * 2 + lax.axis_index("c")

        pltpu.sync_copy(aids_h.at[pl.ds(w * _AG, _AG)], idx_v)

        def abody(i, carry):
            hs = []
            for g in range(_BATCH):
                hs.append(pltpu.async_copy(
                    table_h.at[idx_v.at[i * _BATCH + g]],
                    rows_v.at[pl.ds(g * _GRP, _GRP)], sem))
            for h in hs:
                h.wait()
            pltpu.sync_copy(rows_v,
                            outa_h.at[pl.ds(w * _APW + i * _ROWS, _ROWS)])
            return carry

        lax.fori_loop(0, _AI, abody, 0)

        first8 = w < 8
        cbatches = lax.select(first8, 7, 6)
        cbase_g = lax.select(first8, 56 * w, 48 * w + 64)

        pltpu.sync_copy(cids_h.at[pl.ds(cbase_g, 48)], idx_v.at[pl.ds(0, 48)])

        @pl.when(first8)
        def _():
            pltpu.sync_copy(cids_h.at[pl.ds(cbase_g + 48, 8)],
                            idx_v.at[pl.ds(48, 8)])

        def cbody(i, carry):
            hs = []
            for g in range(_BATCH):
                hs.append(pltpu.async_copy(
                    table_h.at[idx_v.at[i * _BATCH + g]],
                    rows_v.at[pl.ds(g * _GRP, _GRP)], sem))
            base = (cbase_g + i * _BATCH) * _GRP
            pltpu.sync_copy(nums_h.at[pl.ds(base, _ROWS)], nums_v)
            for h in hs:
                h.wait()
            pltpu.sync_copy(rows_v, outc_h.at[pl.ds(base, _ROWS), pl.ds(0, 64)])
            pltpu.sync_copy(nums_v, outc_h.at[pl.ds(base, _ROWS), pl.ds(64, 16)])
            return carry

        lax.fori_loop(0, cbatches, cbody, 0)

    return k(table, cids, aids, nums)


def kernel(reals, card_nums, embed_table, cardIDs, actionIDs):
    cids = cardIDs.astype(jnp.int32).reshape(_CARD_N // _GRP, _GRP)
    aids = actionIDs.astype(jnp.int32).reshape(_ACT_N // _GRP, _GRP)
    nums = card_nums.reshape(_CARD_N, 16)
    out_c, out_a = _sc_prepare(embed_table, cids, aids, nums)
    return (reals, out_c.reshape(_B, 50, 80), out_a.reshape(_B, 20, 512))


# speedup vs baseline: 4.8407x; 1.1768x over previous
import functools

import jax
import jax.numpy as jnp
from jax import lax
from jax.experimental import pallas as pl
from jax.experimental.pallas import tpu as pltpu
from jax.experimental.pallas import tpu_sc as plsc

_GRP = 128
_NW = 32
_B = 4096
_A = 20
_CARD_N = _B * 50
_ACT_N = _B * _A * 8
_BATCH = 8
_ROWS = _BATCH * _GRP
_APW = _ACT_N // _NW
_AG = _APW // _GRP
_AI = _AG // _BATCH


def _sc_prepare(table, cids, aids, nums):
    mesh = plsc.VectorSubcoreMesh(core_axis_name="c", subcore_axis_name="s")

    @functools.partial(
        pl.kernel,
        mesh=mesh,
        out_type=[
            jax.ShapeDtypeStruct((_CARD_N, 80), jnp.float32),
            jax.ShapeDtypeStruct((_ACT_N, 64), jnp.float32),
        ],
        scratch_types=[
            pltpu.VMEM((_AG, _GRP), jnp.int32),
            pltpu.VMEM((_ROWS, 64), jnp.float32),
            pltpu.VMEM((_ROWS, 16), jnp.float32),
            pltpu.SemaphoreType.DMA,
        ],
        compiler_params=pltpu.CompilerParams(use_tc_tiling_on_sc=False),
    )
    def k(table_h, cids_h, aids_h, nums_h, outc_h, outa_h,
          idx_v, rows_v, nums_v, sem):
        w = lax.axis_index("s") * 2 + lax.axis_index("c")

        pltpu.sync_copy(aids_h.at[pl.ds(w * _AG, _AG)], idx_v)

        def abody(i, carry):
            hs = []
            for g in range(_BATCH):
                hs.append(pltpu.async_copy(
                    table_h.at[idx_v.at[i * _BATCH + g]],
                    rows_v.at[pl.ds(g * _GRP, _GRP)], sem))
            for h in hs:
                h.wait()
            pltpu.sync_copy(rows_v,
                            outa_h.at[pl.ds(w * _APW + i * _ROWS, _ROWS)])
            return carry

        lax.fori_loop(0, _AI, abody, 0)

        first8 = w < 8
        cbatches = lax.select(first8, 7, 6)
        cbase_g = lax.select(first8, 56 * w, 48 * w + 64)

        pltpu.sync_copy(cids_h.at[pl.ds(cbase_g, 48)], idx_v.at[pl.ds(0, 48)])

        @pl.when(first8)
        def _():
            pltpu.sync_copy(cids_h.at[pl.ds(cbase_g + 48, 8)],
                            idx_v.at[pl.ds(48, 8)])

        def cbody(i, carry):
            hs = []
            for g in range(_BATCH):
                hs.append(pltpu.async_copy(
                    table_h.at[idx_v.at[i * _BATCH + g]],
                    rows_v.at[pl.ds(g * _GRP, _GRP)], sem))
            base = (cbase_g + i * _BATCH) * _GRP
            pltpu.sync_copy(nums_h.at[pl.ds(base, _ROWS)], nums_v)
            for h in hs:
                h.wait()
            pltpu.sync_copy(rows_v, outc_h.at[pl.ds(base, _ROWS), pl.ds(0, 64)])
            pltpu.sync_copy(nums_v, outc_h.at[pl.ds(base, _ROWS), pl.ds(64, 16)])
            return carry

        lax.fori_loop(0, cbatches, cbody, 0)

    return k(table, cids, aids, nums)


def kernel(reals, card_nums, embed_table, cardIDs, actionIDs):
    cids = cardIDs.astype(jnp.int32).reshape(_CARD_N // _GRP, _GRP)
    aids = actionIDs.astype(jnp.int32).transpose(1, 0, 2).reshape(
        _ACT_N // _GRP, _GRP)
    nums = card_nums.reshape(_CARD_N, 16)
    out_c, out_a = _sc_prepare(embed_table, cids, aids, nums)
    act = out_a.reshape(_A, _B, 512).transpose(1, 0, 2)
    return (reals, out_c.reshape(_B, 50, 80), act)
